# Initial kernel scaffold; baseline (speedup 1.0000x reference)
import jax
import jax.numpy as jnp
from jax.experimental import pallas as pl


def kernel(x, edge_index, W0, b0, attn_l0, attn_r0, W1, b1, attn_l1, attn_r1):
    raise NotImplementedError("write your pallas kernel here")


# speedup vs baseline: 36.9565x; 36.9565x over previous
import functools

import jax
import jax.numpy as jnp
from jax import lax
from jax.experimental import pallas as pl
from jax.experimental.pallas import tpu as pltpu
from jax.experimental.pallas import tpu_sc as plsc

N = 10000
E = 320000
D = 128
H = 8
FH = 16

NC = 2
NS = 16
NW = NC * NS
EPW = E // NW
CH = 80
NCHUNK = EPW // CH
NP = 10240
RPT = NP // NS

_f32 = jnp.float32


def _attn_mat(attn):
    rows = jnp.arange(H * FH)
    cols = rows // FH
    M = jnp.zeros((H * FH, 2 * H), _f32)
    M = M.at[rows, cols].set(attn.reshape(-1))
    M = M.at[rows, cols + H].set(attn.reshape(-1))
    return M


def _expand_mat():
    r = jnp.arange(16)[:, None]
    c = jnp.arange(D)[None, :]
    return (((c // FH) == r) & (r < H)).astype(_f32)


def _tc_prep(h, WT, ALd, ARd):
    B = 2000

    def body(h_ref, w_ref, al_ref, ar_ref, f_ref, el_ref, er_ref):
        f = jnp.dot(h_ref[...], w_ref[...], preferred_element_type=_f32)
        f_ref[...] = f
        el_ref[...] = jnp.dot(f, al_ref[...], preferred_element_type=_f32)
        er_ref[...] = jnp.dot(f, ar_ref[...], preferred_element_type=_f32)

    return pl.pallas_call(
        body,
        grid=(N // B,),
        in_specs=[
            pl.BlockSpec((B, D), lambda i: (i, 0)),
            pl.BlockSpec((D, D), lambda i: (0, 0)),
            pl.BlockSpec((D, 2 * H), lambda i: (0, 0)),
            pl.BlockSpec((D, 2 * H), lambda i: (0, 0)),
        ],
        out_specs=[
            pl.BlockSpec((B, D), lambda i: (i, 0)),
            pl.BlockSpec((B, 2 * H), lambda i: (i, 0)),
            pl.BlockSpec((B, 2 * H), lambda i: (i, 0)),
        ],
        out_shape=[
            jax.ShapeDtypeStruct((N, D), _f32),
            jax.ShapeDtypeStruct((N, 2 * H), _f32),
            jax.ShapeDtypeStruct((N, 2 * H), _f32),
        ],
    )(h, WT, ALd, ARd)


def _tc_mid(dp, mp, b, Eexp, WT, ALd, ARd):
    B = 2000

    def body(dp_ref, mp_ref, b_ref, e_ref, w_ref, al_ref, ar_ref,
             f_ref, el_ref, er_ref):
        d = dp_ref[0] + dp_ref[1]
        m = mp_ref[0] + mp_ref[1]
        invd = 1.0 / (d + 1e-9)
        invdx = jnp.dot(invd, e_ref[...], preferred_element_type=_f32)
        rst = m * invdx + b_ref[...]
        hh = jnp.where(rst >= 0, rst, 0.01 * rst)
        f = jnp.dot(hh, w_ref[...], preferred_element_type=_f32)
        f_ref[...] = f
        el_ref[...] = jnp.dot(f, al_ref[...], preferred_element_type=_f32)
        er_ref[...] = jnp.dot(f, ar_ref[...], preferred_element_type=_f32)

    return pl.pallas_call(
        body,
        grid=(N // B,),
        in_specs=[
            pl.BlockSpec((NC, B, 2 * H), lambda i: (0, i, 0)),
            pl.BlockSpec((NC, B, D), lambda i: (0, i, 0)),
            pl.BlockSpec((1, D), lambda i: (0, 0)),
            pl.BlockSpec((2 * H, D), lambda i: (0, 0)),
            pl.BlockSpec((D, D), lambda i: (0, 0)),
            pl.BlockSpec((D, 2 * H), lambda i: (0, 0)),
            pl.BlockSpec((D, 2 * H), lambda i: (0, 0)),
        ],
        out_specs=[
            pl.BlockSpec((B, D), lambda i: (i, 0)),
            pl.BlockSpec((B, 2 * H), lambda i: (i, 0)),
            pl.BlockSpec((B, 2 * H), lambda i: (i, 0)),
        ],
        out_shape=[
            jax.ShapeDtypeStruct((N, D), _f32),
            jax.ShapeDtypeStruct((N, 2 * H), _f32),
            jax.ShapeDtypeStruct((N, 2 * H), _f32),
        ],
    )(dp, mp, b, Eexp, WT, ALd, ARd)


def _tc_final(dp, mp, b, Eexp):
    B = 2000

    def body(dp_ref, mp_ref, b_ref, e_ref, o_ref):
        d = dp_ref[0] + dp_ref[1]
        m = mp_ref[0] + mp_ref[1]
        invd = 1.0 / (d + 1e-9)
        invdx = jnp.dot(invd, e_ref[...], preferred_element_type=_f32)
        o_ref[...] = m * invdx + b_ref[...]

    return pl.pallas_call(
        body,
        grid=(N // B,),
        in_specs=[
            pl.BlockSpec((NC, B, 2 * H), lambda i: (0, i, 0)),
            pl.BlockSpec((NC, B, D), lambda i: (0, i, 0)),
            pl.BlockSpec((1, D), lambda i: (0, 0)),
            pl.BlockSpec((2 * H, D), lambda i: (0, 0)),
        ],
        out_specs=pl.BlockSpec((B, D), lambda i: (i, 0)),
        out_shape=jax.ShapeDtypeStruct((N, D), _f32),
    )(dp, mp, b, Eexp)


def _edge_pass(src, dst, eld, erd, feat):
    mesh = plsc.VectorSubcoreMesh(core_axis_name="c", subcore_axis_name="s")

    @functools.partial(
        pl.kernel,
        out_type=(
            jax.ShapeDtypeStruct((NC, NP, 2 * H), _f32),
            jax.ShapeDtypeStruct((NC, NP, D), _f32),
        ),
        mesh=mesh,
        compiler_params=pltpu.CompilerParams(
            needs_layout_passes=False, use_tc_tiling_on_sc=False),
        scratch_types=[
            pltpu.VMEM((CH,), jnp.int32),
            pltpu.VMEM((CH,), jnp.int32),
            pltpu.VMEM((CH, 2 * H), _f32),
            pltpu.VMEM((CH, 2 * H), _f32),
            pltpu.VMEM((CH, 2 * H), _f32),
            pltpu.VMEM((CH, D), _f32),
            pltpu.VMEM_SHARED((NP, 2 * H), _f32),
            pltpu.VMEM_SHARED((NP, D), _f32),
        ],
    )
    def kfn(src_h, dst_h, eld_h, erd_h, feat_h, den_h, msg_h,
            idx_s, idx_d, elv, erv, eev, msg, dacc, macc):
        c = lax.axis_index("c")
        s = lax.axis_index("s")
        wid = s * NC + c
        row0 = s * RPT

        def z1(i, _):
            eev[i, :] = jnp.zeros((16,), _f32)
            for jj in range(D // 16):
                msg[i, pl.ds(jj * 16, 16)] = jnp.zeros((16,), _f32)
            return 0
        lax.fori_loop(0, CH, z1, 0)

        for t in range(RPT // CH):
            pltpu.sync_copy(eev, dacc.at[pl.ds(row0 + t * CH, CH)])
            pltpu.sync_copy(msg, macc.at[pl.ds(row0 + t * CH, CH)])
        plsc.subcore_barrier()

        ebase = wid * EPW

        def chunk(j, _):
            base = ebase + j * CH
            pltpu.sync_copy(src_h.at[pl.ds(base, CH)], idx_s)
            pltpu.sync_copy(dst_h.at[pl.ds(base, CH)], idx_d)
            pltpu.sync_copy(eld_h.at[idx_s], elv)
            pltpu.sync_copy(erd_h.at[idx_d], erv)
            pltpu.sync_copy(feat_h.at[idx_s], msg)

            def edge(r, _):
                v = elv[r, :] + erv[r, :]
                v = jnp.where(v >= 0, v, v * 0.2)
                v = jnp.exp(v)
                eev[r, :] = v
                for h in range(H):
                    sc = plsc.load_gather(
                        eev,
                        [jnp.full((16,), r, jnp.int32),
                         jnp.full((16,), h, jnp.int32)],
                    )
                    seg = msg[r, pl.ds(h * FH, FH)]
                    msg[r, pl.ds(h * FH, FH)] = seg * sc
                return 0
            lax.fori_loop(0, CH, edge, 0)

            pltpu.sync_copy(eev, dacc.at[idx_d], add=True)
            pltpu.sync_copy(msg, macc.at[idx_d], add=True)
            return 0
        lax.fori_loop(0, NCHUNK, chunk, 0)

        plsc.subcore_barrier()
        pltpu.sync_copy(dacc.at[pl.ds(row0, RPT)],
                        den_h.at[c, pl.ds(row0, RPT)])
        for t in range(4):
            r0 = row0 + t * (RPT // 4)
            pltpu.sync_copy(macc.at[pl.ds(r0, RPT // 4)],
                            msg_h.at[c, pl.ds(r0, RPT // 4)])

    return kfn(src, dst, eld, erd, feat)


def kernel(x, edge_index, W0, b0, attn_l0, attn_r0, W1, b1, attn_l1, attn_r1):
    src = edge_index[0].astype(jnp.int32)
    dst = edge_index[1].astype(jnp.int32)
    ALd0 = _attn_mat(attn_l0)
    ARd0 = _attn_mat(attn_r0)
    ALd1 = _attn_mat(attn_l1)
    ARd1 = _attn_mat(attn_r1)
    Eexp = _expand_mat()

    f0, el0, er0 = _tc_prep(x, W0.T, ALd0, ARd0)
    dp0, mp0 = _edge_pass(src, dst, el0, er0, f0)
    f1, el1, er1 = _tc_mid(dp0, mp0, b0.reshape(1, D), Eexp, W1.T, ALd1, ARd1)
    dp1, mp1 = _edge_pass(src, dst, el1, er1, f1)
    out = _tc_final(dp1, mp1, b1.reshape(1, D), Eexp)
    return out.reshape(N, H, FH)


# speedup vs baseline: 94.1569x; 2.5478x over previous
import functools

import jax
import jax.numpy as jnp
from jax import lax
from jax.experimental import pallas as pl
from jax.experimental.pallas import tpu as pltpu
from jax.experimental.pallas import tpu_sc as plsc

N = 10000
E = 320000
D = 128
H = 8
FH = 16

NC = 2
NS = 16
NW = NC * NS
EPW = E // NW
CH = 80
NCHUNK = EPW // CH
NP = 10240
RPT = NP // NS

_f32 = jnp.float32


def _attn_mat(attn):
    rows = jnp.arange(H * FH)
    cols = rows // FH
    M = jnp.zeros((H * FH, 2 * H), _f32)
    M = M.at[rows, cols].set(attn.reshape(-1))
    M = M.at[rows, cols + H].set(attn.reshape(-1))
    return M


def _expand_mat():
    r = jnp.arange(16)[:, None]
    c = jnp.arange(D)[None, :]
    return (((c // FH) == r) & (r < H)).astype(_f32)


def _tc_prep(h, WT, ALd, ARd):
    B = 2000

    def body(h_ref, w_ref, al_ref, ar_ref, f_ref, el_ref, er_ref):
        f = jnp.dot(h_ref[...], w_ref[...], preferred_element_type=_f32)
        f_ref[...] = f
        el_ref[...] = jnp.dot(f, al_ref[...], preferred_element_type=_f32)
        er_ref[...] = jnp.dot(f, ar_ref[...], preferred_element_type=_f32)

    return pl.pallas_call(
        body,
        grid=(N // B,),
        in_specs=[
            pl.BlockSpec((B, D), lambda i: (i, 0)),
            pl.BlockSpec((D, D), lambda i: (0, 0)),
            pl.BlockSpec((D, 2 * H), lambda i: (0, 0)),
            pl.BlockSpec((D, 2 * H), lambda i: (0, 0)),
        ],
        out_specs=[
            pl.BlockSpec((B, D), lambda i: (i, 0)),
            pl.BlockSpec((B, 2 * H), lambda i: (i, 0)),
            pl.BlockSpec((B, 2 * H), lambda i: (i, 0)),
        ],
        out_shape=[
            jax.ShapeDtypeStruct((N, D), _f32),
            jax.ShapeDtypeStruct((N, 2 * H), _f32),
            jax.ShapeDtypeStruct((N, 2 * H), _f32),
        ],
    )(h, WT, ALd, ARd)


def _tc_mid(dp, mp, b, Eexp, WT, ALd, ARd):
    B = 2000

    def body(dp_ref, mp_ref, b_ref, e_ref, w_ref, al_ref, ar_ref,
             f_ref, el_ref, er_ref):
        d = dp_ref[0] + dp_ref[1]
        m = mp_ref[0] + mp_ref[1]
        invd = 1.0 / (d + 1e-9)
        invdx = jnp.dot(invd, e_ref[...], preferred_element_type=_f32)
        rst = m * invdx + b_ref[...]
        hh = jnp.where(rst >= 0, rst, 0.01 * rst)
        f = jnp.dot(hh, w_ref[...], preferred_element_type=_f32)
        f_ref[...] = f
        el_ref[...] = jnp.dot(f, al_ref[...], preferred_element_type=_f32)
        er_ref[...] = jnp.dot(f, ar_ref[...], preferred_element_type=_f32)

    return pl.pallas_call(
        body,
        grid=(N // B,),
        in_specs=[
            pl.BlockSpec((NC, B, 2 * H), lambda i: (0, i, 0)),
            pl.BlockSpec((NC, B, D), lambda i: (0, i, 0)),
            pl.BlockSpec((1, D), lambda i: (0, 0)),
            pl.BlockSpec((2 * H, D), lambda i: (0, 0)),
            pl.BlockSpec((D, D), lambda i: (0, 0)),
            pl.BlockSpec((D, 2 * H), lambda i: (0, 0)),
            pl.BlockSpec((D, 2 * H), lambda i: (0, 0)),
        ],
        out_specs=[
            pl.BlockSpec((B, D), lambda i: (i, 0)),
            pl.BlockSpec((B, 2 * H), lambda i: (i, 0)),
            pl.BlockSpec((B, 2 * H), lambda i: (i, 0)),
        ],
        out_shape=[
            jax.ShapeDtypeStruct((N, D), _f32),
            jax.ShapeDtypeStruct((N, 2 * H), _f32),
            jax.ShapeDtypeStruct((N, 2 * H), _f32),
        ],
    )(dp, mp, b, Eexp, WT, ALd, ARd)


def _tc_final(dp, mp, b, Eexp):
    B = 2000

    def body(dp_ref, mp_ref, b_ref, e_ref, o_ref):
        d = dp_ref[0] + dp_ref[1]
        m = mp_ref[0] + mp_ref[1]
        invd = 1.0 / (d + 1e-9)
        invdx = jnp.dot(invd, e_ref[...], preferred_element_type=_f32)
        o_ref[...] = m * invdx + b_ref[...]

    return pl.pallas_call(
        body,
        grid=(N // B,),
        in_specs=[
            pl.BlockSpec((NC, B, 2 * H), lambda i: (0, i, 0)),
            pl.BlockSpec((NC, B, D), lambda i: (0, i, 0)),
            pl.BlockSpec((1, D), lambda i: (0, 0)),
            pl.BlockSpec((2 * H, D), lambda i: (0, 0)),
        ],
        out_specs=pl.BlockSpec((B, D), lambda i: (i, 0)),
        out_shape=jax.ShapeDtypeStruct((N, D), _f32),
    )(dp, mp, b, Eexp)


def _edge_pass(src, dst, eld, erd, feat):
    mesh = plsc.VectorSubcoreMesh(core_axis_name="c", subcore_axis_name="s")

    @functools.partial(
        pl.kernel,
        out_type=(
            jax.ShapeDtypeStruct((NC, NP, 2 * H), _f32),
            jax.ShapeDtypeStruct((NC, NP, D), _f32),
        ),
        mesh=mesh,
        compiler_params=pltpu.CompilerParams(
            needs_layout_passes=False, use_tc_tiling_on_sc=False),
        scratch_types=[
            [pltpu.VMEM((CH,), jnp.int32)] * 2,
            [pltpu.VMEM((CH,), jnp.int32)] * 2,
            [pltpu.VMEM((CH, 2 * H), _f32)] * 2,
            [pltpu.VMEM((CH, 2 * H), _f32)] * 2,
            [pltpu.VMEM((CH, 2 * H), _f32)] * 2,
            [pltpu.VMEM((CH, D), _f32)] * 2,
            [pltpu.SemaphoreType.DMA] * 2,
            [pltpu.SemaphoreType.DMA] * 2,
            pltpu.VMEM_SHARED((NP, 2 * H), _f32),
            pltpu.VMEM_SHARED((NP, D), _f32),
        ],
    )
    def kfn(src_h, dst_h, eld_h, erd_h, feat_h, den_h, msg_h,
            idx_s, idx_d, elv, erv, eev, msg, isem, gsem, dacc, macc):
        c = lax.axis_index("c")
        s = lax.axis_index("s")
        wid = s * NC + c
        row0 = s * RPT
        ebase = wid * EPW

        def z1(i, _):
            eev[0][i, :] = jnp.zeros((16,), _f32)
            for jj in range(D // 16):
                msg[0][i, pl.ds(jj * 16, 16)] = jnp.zeros((16,), _f32)
            return 0
        lax.fori_loop(0, CH, z1, 0)

        for t in range(RPT // CH):
            pltpu.sync_copy(eev[0], dacc.at[pl.ds(row0 + t * CH, CH)])
            pltpu.sync_copy(msg[0], macc.at[pl.ds(row0 + t * CH, CH)])
        plsc.subcore_barrier()

        def issue_idx(j, p):
            base = ebase + j * CH
            pltpu.async_copy(src_h.at[pl.ds(base, CH)], idx_s[p], isem[p])
            pltpu.async_copy(dst_h.at[pl.ds(base, CH)], idx_d[p], isem[p])

        def wait_idx(p):
            pltpu.make_async_copy(src_h.at[pl.ds(0, CH)], idx_s[p],
                                  isem[p]).wait()
            pltpu.make_async_copy(dst_h.at[pl.ds(0, CH)], idx_d[p],
                                  isem[p]).wait()

        def issue_gather(p):
            pltpu.async_copy(eld_h.at[idx_s[p]], elv[p], gsem[p])
            pltpu.async_copy(erd_h.at[idx_d[p]], erv[p], gsem[p])
            pltpu.async_copy(feat_h.at[idx_s[p]], msg[p], gsem[p])

        def wait_gather(p):
            pltpu.make_async_copy(eld_h.at[idx_s[p]], elv[p], gsem[p]).wait()
            pltpu.make_async_copy(erd_h.at[idx_d[p]], erv[p], gsem[p]).wait()
            pltpu.make_async_copy(feat_h.at[idx_s[p]], msg[p], gsem[p]).wait()

        def compute_scatter(p):
            def edge(r, _):
                v = elv[p][r, :] + erv[p][r, :]
                v = jnp.where(v >= 0, v, v * 0.2)
                v = jnp.exp(v)
                eev[p][r, :] = v
                for h in range(H):
                    sc = jnp.take_along_axis(
                        v, jnp.full((16,), h, jnp.int32), axis=0,
                        mode="promise_in_bounds")
                    seg = msg[p][r, pl.ds(h * FH, FH)]
                    msg[p][r, pl.ds(h * FH, FH)] = seg * sc
                return 0
            lax.fori_loop(0, CH, edge, 0)
            pltpu.sync_copy(eev[p], dacc.at[idx_d[p]], add=True)
            pltpu.sync_copy(msg[p], macc.at[idx_d[p]], add=True)

        issue_idx(0, 0)
        issue_idx(1, 1)
        wait_idx(0)
        issue_gather(0)

        def two_chunks(j2, _):
            for b in range(2):
                j = j2 * 2 + b
                p = b
                wait_idx(1 - p)
                issue_gather(1 - p)
                wait_gather(p)
                compute_scatter(p)

                @pl.when(j + 2 < NCHUNK)
                def _():
                    issue_idx(j + 2, p)
            return 0
        lax.fori_loop(0, (NCHUNK - 1) // 2, two_chunks, 0)
        wait_gather(0)
        compute_scatter(0)

        plsc.subcore_barrier()
        pltpu.sync_copy(dacc.at[pl.ds(row0, RPT)],
                        den_h.at[c, pl.ds(row0, RPT)])
        for t in range(RPT // CH):
            r0 = row0 + t * CH
            pltpu.sync_copy(macc.at[pl.ds(r0, CH)],
                            msg_h.at[c, pl.ds(r0, CH)])

    return kfn(src, dst, eld, erd, feat)


def kernel(x, edge_index, W0, b0, attn_l0, attn_r0, W1, b1, attn_l1, attn_r1):
    src = edge_index[0].astype(jnp.int32)
    dst = edge_index[1].astype(jnp.int32)
    ALd0 = _attn_mat(attn_l0)
    ARd0 = _attn_mat(attn_r0)
    ALd1 = _attn_mat(attn_l1)
    ARd1 = _attn_mat(attn_r1)
    Eexp = _expand_mat()

    f0, el0, er0 = _tc_prep(x, W0.T, ALd0, ARd0)
    dp0, mp0 = _edge_pass(src, dst, el0, er0, f0)
    f1, el1, er1 = _tc_mid(dp0, mp0, b0.reshape(1, D), Eexp, W1.T, ALd1, ARd1)
    dp1, mp1 = _edge_pass(src, dst, el1, er1, f1)
    out = _tc_final(dp1, mp1, b1.reshape(1, D), Eexp)
    return out.reshape(N, H, FH)
